# baseline (speedup 1.0000x reference)
import functools

import jax
import jax.numpy as jnp
from jax import lax
from jax.experimental import pallas as pl
from jax.experimental.pallas import tpu as pltpu
from jax.experimental.pallas import tpu_sc as plsc

B = 16384
D = 128
NC = 2
NS = 16
NW = NC * NS
BPW = B // NW
CHUNK = 128
NCHUNK = BPW // CHUNK
LANES = 16
NBUF = 2

_mesh = plsc.VectorSubcoreMesh(core_axis_name="c", subcore_axis_name="s")


@functools.partial(
    pl.kernel,
    mesh=_mesh,
    out_type=jax.ShapeDtypeStruct((B,), jnp.float32),
    scratch_types=[
        pltpu.VMEM((BPW,), jnp.int32),
        pltpu.VMEM((BPW,), jnp.int32),
        pltpu.VMEM((NBUF, CHUNK, D), jnp.float32),
        pltpu.VMEM((NBUF, CHUNK, D), jnp.float32),
        pltpu.VMEM((BPW,), jnp.float32),
        pltpu.SemaphoreType.DMA,
        pltpu.SemaphoreType.DMA,
    ],
)
def _contrastive_sc(w1_hbm, w2_hbm, e1_hbm, e2_hbm, out_hbm,
                    idx1_v, idx2_v, rows1_v, rows2_v, out_v, sem0, sem1):
    wid = lax.axis_index("s") * NC + lax.axis_index("c")
    base = wid * BPW
    sems = (sem0, sem1)
    pltpu.sync_copy(w1_hbm.at[pl.ds(base, BPW)], idx1_v)
    pltpu.sync_copy(w2_hbm.at[pl.ds(base, BPW)], idx2_v)

    def fire(g):
        slot = g % NBUF
        c1 = pltpu.async_copy(
            e1_hbm.at[idx1_v.at[pl.ds(g * CHUNK, CHUNK)]],
            rows1_v.at[slot], sems[slot])
        c2 = pltpu.async_copy(
            e2_hbm.at[idx2_v.at[pl.ds(g * CHUNK, CHUNK)]],
            rows2_v.at[slot], sems[slot])
        return c1, c2

    inflight = {0: fire(0)}

    for g in range(NCHUNK):
        slot = g % NBUF
        if g + 1 < NCHUNK:
            inflight[g + 1] = fire(g + 1)
        c1, c2 = inflight.pop(g)
        c1.wait()
        c2.wait()
        r1 = rows1_v.at[slot]
        r2 = rows2_v.at[slot]

        def group_body(grp, carry, r1=r1, r2=r2):
            lanes = lax.iota(jnp.int32, LANES)
            brv = (0, 8, 4, 12, 2, 10, 6, 14, 1, 9, 5, 13, 3, 11, 7, 15)

            def merge(a, b, w):
                low = (lanes % (2 * w)) < w
                perm = jnp.where(low, lanes + w, lanes - w)
                return jnp.where(low, a, jnp.take(b, perm)) + \
                       jnp.where(low, jnp.take(a, perm), b)

            widths = (8, 4, 2, 1)
            stack = []
            for k in range(LANES):
                i = grp * LANES + brv[k]
                acc = r1[i, pl.ds(0, LANES)] * r2[i, pl.ds(0, LANES)]
                for j in range(1, D // LANES):
                    acc = acc + (r1[i, pl.ds(j * LANES, LANES)]
                                 * r2[i, pl.ds(j * LANES, LANES)])
                v, lvl = acc, 0
                while stack and stack[-1][1] == lvl:
                    u, _ = stack.pop()
                    v = merge(u, v, widths[lvl])
                    lvl += 1
                stack.append((v, lvl))
            tot = stack[0][0]
            out_v[pl.ds(g * CHUNK + grp * LANES, LANES)] = \
                1.0 / (1.0 + jnp.exp(-tot))
            return carry

        lax.fori_loop(0, CHUNK // LANES, group_body, 0)

    pltpu.sync_copy(out_v, out_hbm.at[pl.ds(base, BPW)])


def kernel(word1, word2, emb1_weight, emb2_weight):
    return _contrastive_sc(word1, word2, emb1_weight, emb2_weight)
